# baseline (speedup 1.0000x reference)
import functools

import jax
import jax.numpy as jnp
from jax.experimental import pallas as pl
from jax.experimental.pallas import tpu as pltpu

B, N, D = 8, 2048, 64
_DEFAULT = jax.lax.Precision.DEFAULT
_HIGHEST = jax.lax.Precision.HIGHEST


def _conv_layer_body(x_ref, H_ref, ns_ref, ge_ref,
                     W01_ref, b01_ref, att01_ref,
                     out1_ref, efts_ref):
    j = pl.program_id(1)
    ones = jnp.ones((N, 1), jnp.float32)
    M = N // 2
    h = H_ref[...]
    att = jnp.where(j == 0, att01_ref[0:1, :], att01_ref[1:2, :])
    W = jnp.where(j == 0, W01_ref[:D, :], W01_ref[D:, :])
    b = jnp.where(j == 0, b01_ref[0:1, :], b01_ref[1:2, :])
    xin = jnp.where(j == 0, x_ref[...], out1_ref[...])

    w = jax.nn.sigmoid(ns_ref[...] * att[:, 0:1]
                       + ge_ref[...] * att[:, 1:2])
    xaug = jnp.concatenate([xin * w, w], axis=1)
    eaug = jnp.concatenate(
        [jax.lax.dot_general(
            h[:, :M], xaug, (((0,), (0,)), ((), ())),
            precision=_DEFAULT, preferred_element_type=jnp.float32),
         jax.lax.dot_general(
            h[:, M:], xaug, (((0,), (0,)), ((), ())),
            precision=_DEFAULT, preferred_element_type=jnp.float32)],
        axis=0)
    de = eaug[:, D:D + 1] + 1e-6
    e_fts = eaug[:, :D] / de
    e_proj = jax.lax.dot_general(
        e_fts, W, (((1,), (0,)), ((), ())),
        precision=_DEFAULT, preferred_element_type=jnp.float32) + b
    paug = jnp.concatenate([e_proj, ones], axis=1)
    naug = jnp.concatenate(
        [jax.lax.dot_general(
            h[:M, :], paug, (((1,), (0,)), ((), ())),
            precision=_DEFAULT, preferred_element_type=jnp.float32),
         jax.lax.dot_general(
            h[M:, :], paug, (((1,), (0,)), ((), ())),
            precision=_DEFAULT, preferred_element_type=jnp.float32)],
        axis=0)
    dv = naug[:, D:D + 1] + 1e-6
    node = jax.nn.relu(naug[:, :D] / dv)
    out = xin + node
    efts_ref[...] = e_fts
    out1_ref[...] = out


def _classifier_body(f0_ref, f1_ref, w0_ref, w1_ref, bc1_ref, h_ref):
    k = pl.program_id(0)
    nk = pl.num_programs(0)
    part = jnp.concatenate(
        [jax.lax.dot_general(
            f0_ref[...], w0_ref[...][:, :128], (((1,), (0,)), ((), ())),
            precision=_DEFAULT, preferred_element_type=jnp.float32)
         + jax.lax.dot_general(
            f1_ref[...], w1_ref[...][:, :128], (((1,), (0,)), ((), ())),
            precision=_DEFAULT, preferred_element_type=jnp.float32),
         jax.lax.dot_general(
            f0_ref[...], w0_ref[...][:, 128:], (((1,), (0,)), ((), ())),
            precision=_DEFAULT, preferred_element_type=jnp.float32)
         + jax.lax.dot_general(
            f1_ref[...], w1_ref[...][:, 128:], (((1,), (0,)), ((), ())),
            precision=_DEFAULT, preferred_element_type=jnp.float32)],
        axis=1)

    @pl.when(k == 0)
    def _init():
        h_ref[...] = part

    @pl.when(k > 0)
    def _acc():
        h_ref[...] = h_ref[...] + part

    @pl.when(k == nk - 1)
    def _fin():
        h_ref[...] = jax.nn.relu(h_ref[...] + bc1_ref[...])


def _head_body(h_ref, Wc2_ref, bc2_ref, out_ref):
    logits = jax.lax.dot_general(
        h_ref[...], Wc2_ref[...], (((1,), (0,)), ((), ())),
        precision=_HIGHEST, preferred_element_type=jnp.float32) + bc2_ref[...]
    m = jnp.max(logits, axis=1, keepdims=True)
    z = logits - m
    lse = jnp.log(jnp.sum(jnp.exp(z), axis=1, keepdims=True))
    out_ref[...] = z - lse


@functools.partial(jax.jit, static_argnames=())
def kernel(x, H, node_strength, gloable_efficiency, dist_matrix,
           W0, b0, att0, W1, b1, att1, Wc1, bc1, Wc2, bc2):
    del dist_matrix
    x_r = x.reshape(B * N, D)
    H_r = H.reshape(B * N, N)
    ns_c = node_strength.reshape(B * N, 1)
    ge_c = gloable_efficiency.reshape(B * N, 1)
    att0_r = att0.reshape(1, 2)
    att1_r = att1.reshape(1, 2)
    b0_r = b0.reshape(1, D)
    b1_r = b1.reshape(1, D)
    bc1_r = bc1.reshape(1, 256)
    bc2_r = bc2.reshape(1, 2)

    full = lambda s: pl.BlockSpec(s, lambda b: (0,) * len(s))
    batched = lambda s: pl.BlockSpec(s, lambda b: (b,) + (0,) * (len(s) - 1))

    W01 = jnp.concatenate([W0, W1], axis=0)
    b01 = jnp.concatenate([b0_r, b1_r], axis=0)
    att01 = jnp.concatenate([att0_r, att1_r], axis=0)

    b2 = lambda s: pl.BlockSpec(s, lambda b, j: (b,) + (0,) * (len(s) - 1))
    f2 = lambda s: pl.BlockSpec(s, lambda b, j: (0,) * len(s))

    out1, e_fts = pl.pallas_call(
        _conv_layer_body,
        grid=(B, 2),
        in_specs=[
            b2((N, D)),
            b2((N, N)),
            b2((N, 1)),
            b2((N, 1)),
            f2((2 * D, D)), f2((2, D)), f2((2, 2)),
        ],
        out_specs=[b2((N, D)), b2((N, D))],
        out_shape=[jax.ShapeDtypeStruct((B * N, D), jnp.float32),
                   jax.ShapeDtypeStruct((B * N, D), jnp.float32)],
        compiler_params=pltpu.CompilerParams(
            dimension_semantics=("parallel", "arbitrary"),
            vmem_limit_bytes=100 * 1024 * 1024,
        ),
    )(x_r, H_r, ns_c, ge_c, W01, b01, att01)

    flat = out1.reshape(B, N * D)
    KT = 8192
    HT = KT // 2
    NKT = (N * D) // KT
    fq = lambda i: pl.BlockSpec((B, HT), lambda k, i=i: (0, 2 * k + i))
    wq = lambda i: pl.BlockSpec((HT, 256), lambda k, i=i: (2 * k + i, 0))
    h = pl.pallas_call(
        _classifier_body,
        grid=(NKT,),
        in_specs=[
            fq(0), fq(1), wq(0), wq(1),
            pl.BlockSpec((1, 256), lambda k: (0, 0)),
        ],
        out_specs=pl.BlockSpec((B, 256), lambda k: (0, 0)),
        out_shape=jax.ShapeDtypeStruct((B, 256), jnp.float32),
        compiler_params=pltpu.CompilerParams(
            dimension_semantics=("arbitrary",),
        ),
    )(flat, flat, Wc1, Wc1, bc1_r)

    out = pl.pallas_call(
        _head_body,
        out_shape=jax.ShapeDtypeStruct((B, 2), jnp.float32),
    )(h, Wc2, bc2_r)

    return (out, e_fts.reshape(B, N, D))


# speedup vs baseline: 1.0949x; 1.0949x over previous
import functools

import jax
import jax.numpy as jnp
from jax.experimental import pallas as pl
from jax.experimental.pallas import tpu as pltpu

B, N, D = 8, 2048, 64
_DEFAULT = jax.lax.Precision.DEFAULT
_HIGHEST = jax.lax.Precision.HIGHEST


def _conv_pair_body(x_ref, h0_ref, h1_ref, h2_ref, h3_ref, ns_ref, ge_ref,
                    W0_ref, b0_ref, att0_ref, W1_ref, b1_ref, att1_ref,
                    out1_ref, efts_ref):
    ones = jnp.ones((N, 1), jnp.float32)
    hb = (h0_ref, h1_ref, h2_ref, h3_ref)
    Q = N // 4
    M = N // 2

    def conv(xin, W_r, b_r, att_r):
        att = att_r[...]
        w = jax.nn.sigmoid(ns_ref[...] * att[:, 0:1]
                           + ge_ref[...] * att[:, 1:2])
        xaug = jnp.concatenate([xin * w, w], axis=1)
        def ehalf(lo, hi):
            acc = None
            for i in range(4):
                p = jax.lax.dot_general(
                    hb[i][:, lo:hi], xaug[i * Q:(i + 1) * Q],
                    (((0,), (0,)), ((), ())),
                    precision=_DEFAULT, preferred_element_type=jnp.float32)
                acc = p if acc is None else acc + p
            return acc
        eaug = jnp.concatenate([ehalf(0, M), ehalf(M, N)], axis=0)
        de = eaug[:, D:D + 1] + 1e-6
        e_fts = eaug[:, :D] / de
        e_proj = jax.lax.dot_general(
            e_fts, W_r[...], (((1,), (0,)), ((), ())),
            precision=_DEFAULT, preferred_element_type=jnp.float32) + b_r[...]
        paug = jnp.concatenate([e_proj, ones], axis=1)
        naug = jnp.concatenate(
            [jax.lax.dot_general(
                hb[i][...], paug, (((1,), (0,)), ((), ())),
                precision=_DEFAULT, preferred_element_type=jnp.float32)
             for i in range(4)], axis=0)
        dv = naug[:, D:D + 1] + 1e-6
        node = jax.nn.relu(naug[:, :D] / dv)
        return node, e_fts

    x = x_ref[...]
    node0, _ = conv(x, W0_ref, b0_ref, att0_ref)
    out0 = node0 + x
    node1, e_fts1 = conv(out0, W1_ref, b1_ref, att1_ref)
    efts_ref[...] = e_fts1
    out1_ref[...] = out0 + node1


def _classifier_body(f0_ref, f1_ref, w0_ref, w1_ref, bc1_ref, h_ref):
    k = pl.program_id(0)
    nk = pl.num_programs(0)
    part = jnp.concatenate(
        [jax.lax.dot_general(
            f0_ref[...], w0_ref[...][:, :128], (((1,), (0,)), ((), ())),
            precision=_DEFAULT, preferred_element_type=jnp.float32)
         + jax.lax.dot_general(
            f1_ref[...], w1_ref[...][:, :128], (((1,), (0,)), ((), ())),
            precision=_DEFAULT, preferred_element_type=jnp.float32),
         jax.lax.dot_general(
            f0_ref[...], w0_ref[...][:, 128:], (((1,), (0,)), ((), ())),
            precision=_DEFAULT, preferred_element_type=jnp.float32)
         + jax.lax.dot_general(
            f1_ref[...], w1_ref[...][:, 128:], (((1,), (0,)), ((), ())),
            precision=_DEFAULT, preferred_element_type=jnp.float32)],
        axis=1)

    @pl.when(k == 0)
    def _init():
        h_ref[...] = part

    @pl.when(k > 0)
    def _acc():
        h_ref[...] = h_ref[...] + part

    @pl.when(k == nk - 1)
    def _fin():
        h_ref[...] = jax.nn.relu(h_ref[...] + bc1_ref[...])


def _head_body(h_ref, Wc2_ref, bc2_ref, out_ref):
    logits = jax.lax.dot_general(
        h_ref[...], Wc2_ref[...], (((1,), (0,)), ((), ())),
        precision=_HIGHEST, preferred_element_type=jnp.float32) + bc2_ref[...]
    m = jnp.max(logits, axis=1, keepdims=True)
    z = logits - m
    lse = jnp.log(jnp.sum(jnp.exp(z), axis=1, keepdims=True))
    out_ref[...] = z - lse


@functools.partial(jax.jit, static_argnames=())
def kernel(x, H, node_strength, gloable_efficiency, dist_matrix,
           W0, b0, att0, W1, b1, att1, Wc1, bc1, Wc2, bc2):
    del dist_matrix
    x_r = x.reshape(B * N, D)
    H_r = H.reshape(B * N, N)
    ns_c = node_strength.reshape(B * N, 1)
    ge_c = gloable_efficiency.reshape(B * N, 1)
    att0_r = att0.reshape(1, 2)
    att1_r = att1.reshape(1, 2)
    b0_r = b0.reshape(1, D)
    b1_r = b1.reshape(1, D)
    bc1_r = bc1.reshape(1, 256)
    bc2_r = bc2.reshape(1, 2)

    full = lambda s: pl.BlockSpec(s, lambda b: (0,) * len(s))
    batched = lambda s: pl.BlockSpec(s, lambda b: (b,) + (0,) * (len(s) - 1))

    hband = lambda i: pl.BlockSpec((N // 4, N), lambda b, i=i: (4 * b + i, 0))

    out1, e_fts = pl.pallas_call(
        _conv_pair_body,
        grid=(B,),
        in_specs=[
            batched((N, D)),
            hband(0), hband(1), hband(2), hband(3),
            batched((N, 1)),
            batched((N, 1)),
            full((D, D)), full((1, D)), full((1, 2)),
            full((D, D)), full((1, D)), full((1, 2)),
        ],
        out_specs=[batched((N, D)), batched((N, D))],
        out_shape=[jax.ShapeDtypeStruct((B * N, D), jnp.float32),
                   jax.ShapeDtypeStruct((B * N, D), jnp.float32)],
        compiler_params=pltpu.CompilerParams(
            dimension_semantics=("parallel",),
            vmem_limit_bytes=100 * 1024 * 1024,
        ),
    )(x_r, H_r, H_r, H_r, H_r, ns_c, ge_c, W0, b0_r, att0_r, W1, b1_r, att1_r)

    flat = out1.reshape(B, N * D)
    KT = 8192
    HT = KT // 2
    NKT = (N * D) // KT
    fq = lambda i: pl.BlockSpec((B, HT), lambda k, i=i: (0, 2 * k + i))
    wq = lambda i: pl.BlockSpec((HT, 256), lambda k, i=i: (2 * k + i, 0))
    h = pl.pallas_call(
        _classifier_body,
        grid=(NKT,),
        in_specs=[
            fq(0), fq(1), wq(0), wq(1),
            pl.BlockSpec((1, 256), lambda k: (0, 0)),
        ],
        out_specs=pl.BlockSpec((B, 256), lambda k: (0, 0)),
        out_shape=jax.ShapeDtypeStruct((B, 256), jnp.float32),
        compiler_params=pltpu.CompilerParams(
            dimension_semantics=("arbitrary",),
        ),
    )(flat, flat, Wc1, Wc1, bc1_r)

    out = pl.pallas_call(
        _head_body,
        out_shape=jax.ShapeDtypeStruct((B, 2), jnp.float32),
    )(h, Wc2, bc2_r)

    return (out, e_fts.reshape(B, N, D))
